# Initial kernel scaffold; baseline (speedup 1.0000x reference)
import jax
import jax.numpy as jnp
from jax.experimental import pallas as pl


def kernel(x, edge_index, W1, a_src1, a_dst1, b1, W2, a_src2, a_dst2, b2):
    raise NotImplementedError("write your pallas kernel here")


# speedup vs baseline: 47.9459x; 47.9459x over previous
import functools

import jax
import jax.numpy as jnp
from jax import lax
from jax.experimental import pallas as pl
from jax.experimental.pallas import tpu as pltpu
from jax.experimental.pallas import tpu_sc as plsc

N = 10000
D = 128
H1, C1 = 8, 8
F1 = H1 * C1
NCLS = 7
E = 320000

NT = 10112
BLK = 1264
NW = 32
K = 128
E_PAD = ((E + K * NW - 1) // (K * NW)) * (K * NW)
CPW = E_PAD // (K * NW)
RPS = NT // 16
ROW = 128


def _tc1_body(x_ref, w1_ref, ms_ref, md_ref, rexp_ref, stab_ref, sinit_ref):
    x = x_ref[...]
    h = jnp.dot(x, w1_ref[...], preferred_element_type=jnp.float32)
    a_s = jnp.dot(h, ms_ref[...], preferred_element_type=jnp.float32)
    a_d = jnp.dot(h, md_ref[...], preferred_element_type=jnp.float32)
    z8 = jnp.zeros((x.shape[0], 8), jnp.float32)
    z40 = jnp.zeros((x.shape[0], 40), jnp.float32)
    stab_ref[...] = jnp.concatenate([h, a_s, z8, a_d, z40], axis=1)
    e = a_s + a_d
    ee = jnp.exp(jnp.maximum(e, 0.2 * e))
    ee64 = jnp.dot(ee, rexp_ref[...], preferred_element_type=jnp.float32)
    sinit_ref[...] = jnp.concatenate([h * ee64, ee, z8], axis=1)


def _tc1(x_pad, W1, Ms, Md, Rexp):
    return pl.pallas_call(
        _tc1_body,
        grid=(NT // BLK,),
        in_specs=[
            pl.BlockSpec((BLK, D), lambda i: (i, 0)),
            pl.BlockSpec((D, F1), lambda i: (0, 0)),
            pl.BlockSpec((F1, 8), lambda i: (0, 0)),
            pl.BlockSpec((F1, 8), lambda i: (0, 0)),
            pl.BlockSpec((8, F1), lambda i: (0, 0)),
        ],
        out_specs=[
            pl.BlockSpec((BLK, ROW), lambda i: (i, 0)),
            pl.BlockSpec((BLK, 80), lambda i: (i, 0)),
        ],
        out_shape=[
            jax.ShapeDtypeStruct((NT, ROW), jnp.float32),
            jax.ShapeDtypeStruct((NT, 80), jnp.float32),
        ],
    )(x_pad, W1, Ms, Md, Rexp)


def _sc1_body(src_hbm, dst_hbm, stab_hbm, out_hbm,
              sidx, didx, sbuf, dbuf, obuf, acc, sem1, sem2):
    cid = lax.axis_index("c")
    sid = lax.axis_index("s")
    wid = sid * 2 + cid
    z16 = jnp.zeros((16,), jnp.float32)

    def zrow(e, c):
        for c8 in range(8):
            obuf[e, pl.ds(c8 * 16, 16)] = z16
        return c

    lax.fori_loop(0, K, zrow, 0)
    for i in range(RPS // K):
        pltpu.sync_copy(obuf, acc.at[pl.ds(sid * RPS + i * K, K)])
    rem = RPS % K
    if rem:
        pltpu.sync_copy(obuf.at[pl.ds(0, rem)],
                        acc.at[pl.ds(sid * RPS + (RPS // K) * K, rem)])
    plsc.subcore_barrier()

    lane = lax.iota(jnp.int32, 16)
    g01 = lane >> 3
    g23 = g01 + 2
    g45 = g01 + 4
    g67 = g01 + 6

    def chunk(j, carry):
        base = (wid * CPW + j) * K
        pltpu.sync_copy(src_hbm.at[pl.ds(base, K)], sidx)
        pltpu.sync_copy(dst_hbm.at[pl.ds(base, K)], didx)
        cp1 = pltpu.async_copy(stab_hbm.at[sidx], sbuf, sem1)
        cp2 = pltpu.async_copy(stab_hbm.at[didx], dbuf, sem2)
        cp1.wait()
        cp2.wait()

        def edge(e, c2):
            s4 = sbuf[e, pl.ds(64, 16)]
            d4 = dbuf[e, pl.ds(80, 16)]
            t = s4 + d4
            ee = jnp.exp(jnp.maximum(t, 0.2 * t))
            m01 = jnp.take_along_axis(ee, g01, axis=0,
                                      mode="promise_in_bounds")
            m23 = jnp.take_along_axis(ee, g23, axis=0,
                                      mode="promise_in_bounds")
            m45 = jnp.take_along_axis(ee, g45, axis=0,
                                      mode="promise_in_bounds")
            m67 = jnp.take_along_axis(ee, g67, axis=0,
                                      mode="promise_in_bounds")
            obuf[e, pl.ds(0, 16)] = sbuf[e, pl.ds(0, 16)] * m01
            obuf[e, pl.ds(16, 16)] = sbuf[e, pl.ds(16, 16)] * m23
            obuf[e, pl.ds(32, 16)] = sbuf[e, pl.ds(32, 16)] * m45
            obuf[e, pl.ds(48, 16)] = sbuf[e, pl.ds(48, 16)] * m67
            obuf[e, pl.ds(64, 16)] = jnp.where(lane < 8, ee, 0.0)
            return c2

        lax.fori_loop(0, K, edge, 0, unroll=4)
        pltpu.sync_copy(obuf, acc.at[didx], add=True)
        return carry

    lax.fori_loop(0, CPW, chunk, 0)
    plsc.subcore_barrier()
    pltpu.sync_copy(acc.at[pl.ds(sid * RPS, RPS)],
                    out_hbm.at[cid, pl.ds(sid * RPS, RPS)])


def _sc1(src_p, dst_p, stab):
    mesh = plsc.VectorSubcoreMesh(core_axis_name="c", subcore_axis_name="s")
    f = functools.partial(
        pl.kernel,
        out_type=jax.ShapeDtypeStruct((2, NT, ROW), jnp.float32),
        mesh=mesh,
        scratch_types=[
            pltpu.VMEM((K,), jnp.int32),
            pltpu.VMEM((K,), jnp.int32),
            pltpu.VMEM((K, ROW), jnp.float32),
            pltpu.VMEM((K, ROW), jnp.float32),
            pltpu.VMEM((K, ROW), jnp.float32),
            pltpu.MemorySpace.VMEM_SHARED((NT, ROW), jnp.float32),
            pltpu.SemaphoreType.DMA,
            pltpu.SemaphoreType.DMA,
        ],
    )(_sc1_body)
    return f(src_p, dst_p, stab)


def _tc2_body(acc_ref, sinit_ref, rexp_ref, w2t_ref, b1_ref,
              tab2_ref, sinit2_ref):
    a = acc_ref[0] + acc_ref[1]
    si = sinit_ref[...]
    num = a[:, 0:F1] + si[:, 0:F1]
    den = a[:, F1:F1 + 8] + si[:, F1:F1 + 8]
    dinv = 1.0 / (den + 1e-16)
    d64 = jnp.dot(dinv, rexp_ref[...], preferred_element_type=jnp.float32)
    out1 = num * d64 + b1_ref[...]
    x2 = jnp.where(out1 > 0, out1, jnp.exp(out1) - 1.0)
    t2 = jnp.dot(x2, w2t_ref[...], preferred_element_type=jnp.float32)
    tab2_ref[...] = t2
    a_s = t2[:, 7:8]
    a_d = t2[:, 8:9]
    e = a_s + a_d
    ee = jnp.exp(jnp.maximum(e, 0.2 * e))
    z = jnp.zeros((t2.shape[0], 8), jnp.float32)
    sinit2_ref[...] = jnp.concatenate([t2[:, 0:7] * ee, ee, z], axis=1)


def _tc2(acc1, sinit1, Rexp, W2tab, b1row):
    return pl.pallas_call(
        _tc2_body,
        grid=(NT // BLK,),
        in_specs=[
            pl.BlockSpec((2, BLK, ROW), lambda i: (0, i, 0)),
            pl.BlockSpec((BLK, 80), lambda i: (i, 0)),
            pl.BlockSpec((8, F1), lambda i: (0, 0)),
            pl.BlockSpec((F1, ROW), lambda i: (0, 0)),
            pl.BlockSpec((1, F1), lambda i: (0, 0)),
        ],
        out_specs=[
            pl.BlockSpec((BLK, ROW), lambda i: (i, 0)),
            pl.BlockSpec((BLK, 16), lambda i: (i, 0)),
        ],
        out_shape=[
            jax.ShapeDtypeStruct((NT, ROW), jnp.float32),
            jax.ShapeDtypeStruct((NT, 16), jnp.float32),
        ],
    )(acc1, sinit1, Rexp, W2tab, b1row)


def _sc2_body(src_hbm, dst_hbm, tab_hbm, out_hbm,
              sidx, didx, sbuf, dbuf, obuf, acc, sem1, sem2):
    cid = lax.axis_index("c")
    sid = lax.axis_index("s")
    wid = sid * 2 + cid
    z16 = jnp.zeros((16,), jnp.float32)

    def zrow(e, c):
        for c8 in range(8):
            obuf[e, pl.ds(c8 * 16, 16)] = z16
        return c

    lax.fori_loop(0, K, zrow, 0)
    for i in range(RPS // K):
        pltpu.sync_copy(obuf, acc.at[pl.ds(sid * RPS + i * K, K)])
    rem = RPS % K
    if rem:
        pltpu.sync_copy(obuf.at[pl.ds(0, rem)],
                        acc.at[pl.ds(sid * RPS + (RPS // K) * K, rem)])
    plsc.subcore_barrier()

    lane = lax.iota(jnp.int32, 16)
    c7 = jnp.full((16,), 7, jnp.int32)
    c8v = jnp.full((16,), 8, jnp.int32)

    def chunk(j, carry):
        base = (wid * CPW + j) * K
        pltpu.sync_copy(src_hbm.at[pl.ds(base, K)], sidx)
        pltpu.sync_copy(dst_hbm.at[pl.ds(base, K)], didx)
        cp1 = pltpu.async_copy(tab_hbm.at[sidx], sbuf, sem1)
        cp2 = pltpu.async_copy(tab_hbm.at[didx], dbuf, sem2)
        cp1.wait()
        cp2.wait()

        def edge(e, c2):
            srow = sbuf[e, pl.ds(0, 16)]
            drow = dbuf[e, pl.ds(0, 16)]
            a_s = jnp.take_along_axis(srow, c7, axis=0,
                                      mode="promise_in_bounds")
            a_d = jnp.take_along_axis(drow, c8v, axis=0,
                                      mode="promise_in_bounds")
            t = a_s + a_d
            ee = jnp.exp(jnp.maximum(t, 0.2 * t))
            o = jnp.where(lane < 7, srow * ee, jnp.where(lane == 7, ee, 0.0))
            obuf[e, pl.ds(0, 16)] = o
            return c2

        lax.fori_loop(0, K, edge, 0, unroll=8)
        pltpu.sync_copy(obuf, acc.at[didx], add=True)
        return carry

    lax.fori_loop(0, CPW, chunk, 0)
    plsc.subcore_barrier()
    pltpu.sync_copy(acc.at[pl.ds(sid * RPS, RPS)],
                    out_hbm.at[cid, pl.ds(sid * RPS, RPS)])


def _sc2(src_p, dst_p, tab2):
    mesh = plsc.VectorSubcoreMesh(core_axis_name="c", subcore_axis_name="s")
    f = functools.partial(
        pl.kernel,
        out_type=jax.ShapeDtypeStruct((2, NT, ROW), jnp.float32),
        mesh=mesh,
        scratch_types=[
            pltpu.VMEM((K,), jnp.int32),
            pltpu.VMEM((K,), jnp.int32),
            pltpu.VMEM((K, ROW), jnp.float32),
            pltpu.VMEM((K, ROW), jnp.float32),
            pltpu.VMEM((K, ROW), jnp.float32),
            pltpu.MemorySpace.VMEM_SHARED((NT, ROW), jnp.float32),
            pltpu.SemaphoreType.DMA,
            pltpu.SemaphoreType.DMA,
        ],
    )(_sc2_body)
    return f(src_p, dst_p, tab2)


def _tc3_body(acc_ref, sinit_ref, b2_ref, out_ref):
    a = acc_ref[0] + acc_ref[1]
    si = sinit_ref[...]
    num = a[:, 0:NCLS] + si[:, 0:NCLS]
    den = a[:, NCLS:NCLS + 1] + si[:, NCLS:NCLS + 1]
    logits = num / (den + 1e-16) + b2_ref[...]
    m = jnp.max(logits, axis=1, keepdims=True)
    s = logits - m
    lse = jnp.log(jnp.sum(jnp.exp(s), axis=1, keepdims=True))
    out_ref[...] = jnp.concatenate(
        [s - lse, jnp.zeros((a.shape[0], 1), jnp.float32)], axis=1)


def _tc3(acc2, sinit2, b2row):
    return pl.pallas_call(
        _tc3_body,
        grid=(NT // BLK,),
        in_specs=[
            pl.BlockSpec((2, BLK, ROW), lambda i: (0, i, 0)),
            pl.BlockSpec((BLK, 16), lambda i: (i, 0)),
            pl.BlockSpec((1, NCLS), lambda i: (0, 0)),
        ],
        out_specs=pl.BlockSpec((BLK, 8), lambda i: (i, 0)),
        out_shape=jax.ShapeDtypeStruct((NT, 8), jnp.float32),
    )(acc2, sinit2, b2row)


def kernel(x, edge_index, W1, a_src1, a_dst1, b1, W2, a_src2, a_dst2, b2):
    f32 = jnp.float32
    x_pad = jnp.concatenate([x, jnp.zeros((NT - N, D), f32)], axis=0)

    eye8 = jnp.eye(8, dtype=f32)
    Rexp = jnp.kron(eye8, jnp.ones((1, 8), f32))
    Ms = Rexp.T * a_src1.reshape(F1, 1)
    Md = Rexp.T * a_dst1.reshape(F1, 1)

    w2s = W2 @ a_src2.reshape(NCLS, 1)
    w2d = W2 @ a_dst2.reshape(NCLS, 1)
    W2tab = jnp.concatenate(
        [W2, w2s, w2d, jnp.zeros((F1, ROW - 9), f32)], axis=1)

    src = edge_index[0].astype(jnp.int32)
    dst = edge_index[1].astype(jnp.int32)
    pad_idx = N + (jnp.arange(E_PAD - E, dtype=jnp.int32) % 16)
    src_p = jnp.concatenate([src, pad_idx])
    dst_p = jnp.concatenate([dst, pad_idx])
    src_p, dst_p = jax.lax.optimization_barrier((src_p, dst_p))

    b1row = b1.reshape(1, F1)
    b2row = b2.reshape(1, NCLS)

    stab, sinit1 = _tc1(x_pad, W1, Ms, Md, Rexp)
    acc1 = _sc1(src_p, dst_p, stab)
    tab2, sinit2 = _tc2(acc1, sinit1, Rexp, W2tab, b1row)
    acc2 = _sc2(src_p, dst_p, tab2)
    out = _tc3(acc2, sinit2, b2row)
    return out[:N, :NCLS]


# speedup vs baseline: 74.1875x; 1.5473x over previous
import functools

import jax
import jax.numpy as jnp
from jax import lax
from jax.experimental import pallas as pl
from jax.experimental.pallas import tpu as pltpu
from jax.experimental.pallas import tpu_sc as plsc

N = 10000
D = 128
H1, C1 = 8, 8
F1 = H1 * C1
NCLS = 7
E = 320000

NT = 10112
BLK = 1264
NW = 32
K = 64
_CPW_RAW = -(-E // (K * NW))
CPW = _CPW_RAW + (_CPW_RAW % 2)
E_PAD = CPW * K * NW
RPS = NT // 16
ROW = 128


def _tc1_body(x_ref, w1_ref, ms_ref, md_ref, rexp_ref, stab_ref, sinit_ref):
    x = x_ref[...]
    h = jnp.dot(x, w1_ref[...], preferred_element_type=jnp.float32)
    a_s = jnp.dot(h, ms_ref[...], preferred_element_type=jnp.float32)
    a_d = jnp.dot(h, md_ref[...], preferred_element_type=jnp.float32)
    z8 = jnp.zeros((x.shape[0], 8), jnp.float32)
    z40 = jnp.zeros((x.shape[0], 40), jnp.float32)
    stab_ref[...] = jnp.concatenate([h, a_s, z8, a_d, z40], axis=1)
    e = a_s + a_d
    ee = jnp.exp(jnp.maximum(e, 0.2 * e))
    ee64 = jnp.dot(ee, rexp_ref[...], preferred_element_type=jnp.float32)
    sinit_ref[...] = jnp.concatenate([h * ee64, ee, z8], axis=1)


def _tc1(x_pad, W1, Ms, Md, Rexp):
    return pl.pallas_call(
        _tc1_body,
        grid=(NT // BLK,),
        in_specs=[
            pl.BlockSpec((BLK, D), lambda i: (i, 0)),
            pl.BlockSpec((D, F1), lambda i: (0, 0)),
            pl.BlockSpec((F1, 8), lambda i: (0, 0)),
            pl.BlockSpec((F1, 8), lambda i: (0, 0)),
            pl.BlockSpec((8, F1), lambda i: (0, 0)),
        ],
        out_specs=[
            pl.BlockSpec((BLK, ROW), lambda i: (i, 0)),
            pl.BlockSpec((BLK, 80), lambda i: (i, 0)),
        ],
        out_shape=[
            jax.ShapeDtypeStruct((NT, ROW), jnp.float32),
            jax.ShapeDtypeStruct((NT, 80), jnp.float32),
        ],
    )(x_pad, W1, Ms, Md, Rexp)


def _make_sc_body(edge_compute, unroll):
    def body(src_hbm, dst_hbm, stab_hbm, out_hbm,
             si0, si1, di0, di1, sx0, sx1, sbuf0, dbuf0, sbuf1, dbuf1,
             obuf0, obuf1, acc,
             is0, is1, gs0, gd0, gs1, gd1, ss0, ss1):
        cid = lax.axis_index("c")
        sid = lax.axis_index("s")
        wid = sid * 2 + cid
        z16 = jnp.zeros((16,), jnp.float32)

        def zrow(e, c):
            for c8 in range(8):
                obuf0[e, pl.ds(c8 * 16, 16)] = z16
                obuf1[e, pl.ds(c8 * 16, 16)] = z16
            return c

        lax.fori_loop(0, K, zrow, 0)
        for i in range(RPS // K):
            pltpu.sync_copy(obuf0, acc.at[pl.ds(sid * RPS + i * K, K)])
        rem = RPS % K
        if rem:
            pltpu.sync_copy(obuf0.at[pl.ds(0, rem)],
                            acc.at[pl.ds(sid * RPS + (RPS // K) * K, rem)])
        plsc.subcore_barrier()

        base0 = wid * CPW * K

        def start_idx(j, si, di, isem):
            pltpu.async_copy(src_hbm.at[pl.ds(base0 + j * K, K)], si, isem)
            pltpu.async_copy(dst_hbm.at[pl.ds(base0 + j * K, K)], di, isem)

        def wait_idx(j, si, di, isem):
            pltpu.make_async_copy(
                src_hbm.at[pl.ds(base0 + j * K, K)], si, isem).wait()
            pltpu.make_async_copy(
                dst_hbm.at[pl.ds(base0 + j * K, K)], di, isem).wait()

        def start_g(si, di, sb, db, gs, gd):
            pltpu.async_copy(stab_hbm.at[si], sb, gs)
            pltpu.async_copy(stab_hbm.at[di], db, gd)

        def wait_g(si, di, sb, db, gs, gd):
            pltpu.make_async_copy(stab_hbm.at[si], sb, gs).wait()
            pltpu.make_async_copy(stab_hbm.at[di], db, gd).wait()

        def compute(sb, db, ob):
            def edge(e, c2):
                edge_compute(sb, db, ob, e)
                return c2

            lax.fori_loop(0, K, edge, 0, unroll=unroll)

        def wait_s(ob, sx, ss):
            pltpu.make_async_copy(ob, acc.at[sx], ss).wait()

        SI = (si0, si1)
        DI = (di0, di1)
        SX = (sx0, sx1)
        SB = (sbuf0, sbuf1)
        DB = (dbuf0, dbuf1)
        OB = (obuf0, obuf1)
        IS = (is0, is1)
        GS = (gs0, gs1)
        GD = (gd0, gd1)
        SS = (ss0, ss1)

        start_idx(0, si0, di0, is0)
        wait_idx(0, si0, di0, is0)
        start_g(si0, di0, sbuf0, dbuf0, gs0, gd0)
        start_idx(1, si1, di1, is1)

        def pipe(i, carry):
            j0 = i * 2
            for b in range(2):
                c = j0 + b
                o = 1 - b

                @pl.when(c + 1 < CPW)
                def _():
                    wait_idx(c + 1, SI[o], DI[o], IS[o])
                    start_g(SI[o], DI[o], SB[o], DB[o], GS[o], GD[o])

                wait_g(SI[b], DI[b], SB[b], DB[b], GS[b], GD[b])

                @pl.when(c >= 2)
                def _():
                    wait_s(OB[b], SX[b], SS[b])

                compute(SB[b], DB[b], OB[b])
                for q in range(K // 16):
                    SX[b][pl.ds(q * 16, 16)] = DI[b][pl.ds(q * 16, 16)]
                pltpu.async_copy(OB[b], acc.at[SX[b]], SS[b], add=True)

                @pl.when(c + 2 < CPW)
                def _():
                    start_idx(c + 2, SI[b], DI[b], IS[b])

            return carry

        lax.fori_loop(0, CPW // 2, pipe, 0)
        wait_s(obuf0, sx0, ss0)
        wait_s(obuf1, sx1, ss1)
        plsc.subcore_barrier()
        pltpu.sync_copy(acc.at[pl.ds(sid * RPS, RPS)],
                        out_hbm.at[cid, pl.ds(sid * RPS, RPS)])

    return body


def _sc_call(body):
    mesh = plsc.VectorSubcoreMesh(core_axis_name="c", subcore_axis_name="s")
    return functools.partial(
        pl.kernel,
        out_type=jax.ShapeDtypeStruct((2, NT, ROW), jnp.float32),
        mesh=mesh,
        scratch_types=[
            pltpu.VMEM((K,), jnp.int32),
            pltpu.VMEM((K,), jnp.int32),
            pltpu.VMEM((K,), jnp.int32),
            pltpu.VMEM((K,), jnp.int32),
            pltpu.VMEM((K,), jnp.int32),
            pltpu.VMEM((K,), jnp.int32),
            pltpu.VMEM((K, ROW), jnp.float32),
            pltpu.VMEM((K, ROW), jnp.float32),
            pltpu.VMEM((K, ROW), jnp.float32),
            pltpu.VMEM((K, ROW), jnp.float32),
            pltpu.VMEM((K, ROW), jnp.float32),
            pltpu.VMEM((K, ROW), jnp.float32),
            pltpu.MemorySpace.VMEM_SHARED((NT, ROW), jnp.float32),
            pltpu.SemaphoreType.DMA,
            pltpu.SemaphoreType.DMA,
            pltpu.SemaphoreType.DMA,
            pltpu.SemaphoreType.DMA,
            pltpu.SemaphoreType.DMA,
            pltpu.SemaphoreType.DMA,
            pltpu.SemaphoreType.DMA,
            pltpu.SemaphoreType.DMA,
        ],
    )(body)


def _edge1(lane, g01, g23, g45, g67):
    def f(sb, db, ob, e):
        s4 = sb[e, pl.ds(64, 16)]
        d4 = db[e, pl.ds(80, 16)]
        t = s4 + d4
        ee = jnp.exp(jnp.maximum(t, 0.2 * t))
        m01 = jnp.take_along_axis(ee, g01, axis=0, mode="promise_in_bounds")
        m23 = jnp.take_along_axis(ee, g23, axis=0, mode="promise_in_bounds")
        m45 = jnp.take_along_axis(ee, g45, axis=0, mode="promise_in_bounds")
        m67 = jnp.take_along_axis(ee, g67, axis=0, mode="promise_in_bounds")
        ob[e, pl.ds(0, 16)] = sb[e, pl.ds(0, 16)] * m01
        ob[e, pl.ds(16, 16)] = sb[e, pl.ds(16, 16)] * m23
        ob[e, pl.ds(32, 16)] = sb[e, pl.ds(32, 16)] * m45
        ob[e, pl.ds(48, 16)] = sb[e, pl.ds(48, 16)] * m67
        ob[e, pl.ds(64, 16)] = jnp.where(lane < 8, ee, 0.0)
    return f


def _sc1_body(src_hbm, dst_hbm, stab_hbm, out_hbm, *rest):
    lane = lax.iota(jnp.int32, 16)
    g01 = lane >> 3
    body = _make_sc_body(_edge1(lane, g01, g01 + 2, g01 + 4, g01 + 6),
                         unroll=4)
    return body(src_hbm, dst_hbm, stab_hbm, out_hbm, *rest)


def _sc1(src_p, dst_p, stab):
    return _sc_call(_sc1_body)(src_p, dst_p, stab)


def _tc2_body(acc_ref, sinit_ref, rexp_ref, w2t_ref, b1_ref,
              tab2_ref, sinit2_ref):
    a = acc_ref[0] + acc_ref[1]
    si = sinit_ref[...]
    num = a[:, 0:F1] + si[:, 0:F1]
    den = a[:, F1:F1 + 8] + si[:, F1:F1 + 8]
    dinv = 1.0 / (den + 1e-16)
    d64 = jnp.dot(dinv, rexp_ref[...], preferred_element_type=jnp.float32)
    out1 = num * d64 + b1_ref[...]
    x2 = jnp.where(out1 > 0, out1, jnp.exp(out1) - 1.0)
    t2 = jnp.dot(x2, w2t_ref[...], preferred_element_type=jnp.float32)
    tab2_ref[...] = t2
    a_s = t2[:, 7:8]
    a_d = t2[:, 8:9]
    e = a_s + a_d
    ee = jnp.exp(jnp.maximum(e, 0.2 * e))
    z = jnp.zeros((t2.shape[0], 8), jnp.float32)
    sinit2_ref[...] = jnp.concatenate([t2[:, 0:7] * ee, ee, z], axis=1)


def _tc2(acc1, sinit1, Rexp, W2tab, b1row):
    return pl.pallas_call(
        _tc2_body,
        grid=(NT // BLK,),
        in_specs=[
            pl.BlockSpec((2, BLK, ROW), lambda i: (0, i, 0)),
            pl.BlockSpec((BLK, 80), lambda i: (i, 0)),
            pl.BlockSpec((8, F1), lambda i: (0, 0)),
            pl.BlockSpec((F1, ROW), lambda i: (0, 0)),
            pl.BlockSpec((1, F1), lambda i: (0, 0)),
        ],
        out_specs=[
            pl.BlockSpec((BLK, ROW), lambda i: (i, 0)),
            pl.BlockSpec((BLK, 16), lambda i: (i, 0)),
        ],
        out_shape=[
            jax.ShapeDtypeStruct((NT, ROW), jnp.float32),
            jax.ShapeDtypeStruct((NT, 16), jnp.float32),
        ],
    )(acc1, sinit1, Rexp, W2tab, b1row)


def _edge2(lane, c7, c8v):
    def f(sb, db, ob, e):
        srow = sb[e, pl.ds(0, 16)]
        drow = db[e, pl.ds(0, 16)]
        a_s = jnp.take_along_axis(srow, c7, axis=0, mode="promise_in_bounds")
        a_d = jnp.take_along_axis(drow, c8v, axis=0, mode="promise_in_bounds")
        t = a_s + a_d
        ee = jnp.exp(jnp.maximum(t, 0.2 * t))
        o = jnp.where(lane < 7, srow * ee, jnp.where(lane == 7, ee, 0.0))
        ob[e, pl.ds(0, 16)] = o
    return f


def _sc2_body(src_hbm, dst_hbm, tab_hbm, out_hbm, *rest):
    lane = lax.iota(jnp.int32, 16)
    c7 = jnp.full((16,), 7, jnp.int32)
    c8v = jnp.full((16,), 8, jnp.int32)
    body = _make_sc_body(_edge2(lane, c7, c8v), unroll=8)
    return body(src_hbm, dst_hbm, tab_hbm, out_hbm, *rest)


def _sc2(src_p, dst_p, tab2):
    return _sc_call(_sc2_body)(src_p, dst_p, tab2)


def _tc3_body(acc_ref, sinit_ref, b2_ref, out_ref):
    a = acc_ref[0] + acc_ref[1]
    si = sinit_ref[...]
    num = a[:, 0:NCLS] + si[:, 0:NCLS]
    den = a[:, NCLS:NCLS + 1] + si[:, NCLS:NCLS + 1]
    logits = num / (den + 1e-16) + b2_ref[...]
    m = jnp.max(logits, axis=1, keepdims=True)
    s = logits - m
    lse = jnp.log(jnp.sum(jnp.exp(s), axis=1, keepdims=True))
    out_ref[...] = jnp.concatenate(
        [s - lse, jnp.zeros((a.shape[0], 1), jnp.float32)], axis=1)


def _tc3(acc2, sinit2, b2row):
    return pl.pallas_call(
        _tc3_body,
        grid=(NT // BLK,),
        in_specs=[
            pl.BlockSpec((2, BLK, ROW), lambda i: (0, i, 0)),
            pl.BlockSpec((BLK, 16), lambda i: (i, 0)),
            pl.BlockSpec((1, NCLS), lambda i: (0, 0)),
        ],
        out_specs=pl.BlockSpec((BLK, 8), lambda i: (i, 0)),
        out_shape=jax.ShapeDtypeStruct((NT, 8), jnp.float32),
    )(acc2, sinit2, b2row)


def kernel(x, edge_index, W1, a_src1, a_dst1, b1, W2, a_src2, a_dst2, b2):
    f32 = jnp.float32
    x_pad = jnp.concatenate([x, jnp.zeros((NT - N, D), f32)], axis=0)

    eye8 = jnp.eye(8, dtype=f32)
    Rexp = jnp.kron(eye8, jnp.ones((1, 8), f32))
    Ms = Rexp.T * a_src1.reshape(F1, 1)
    Md = Rexp.T * a_dst1.reshape(F1, 1)

    w2s = W2 @ a_src2.reshape(NCLS, 1)
    w2d = W2 @ a_dst2.reshape(NCLS, 1)
    W2tab = jnp.concatenate(
        [W2, w2s, w2d, jnp.zeros((F1, ROW - 9), f32)], axis=1)

    src = edge_index[0].astype(jnp.int32)
    dst = edge_index[1].astype(jnp.int32)
    pad_idx = N + (jnp.arange(E_PAD - E, dtype=jnp.int32) % 16)
    src_p = jnp.concatenate([src, pad_idx])
    dst_p = jnp.concatenate([dst, pad_idx])
    src_p, dst_p = jax.lax.optimization_barrier((src_p, dst_p))

    b1row = b1.reshape(1, F1)
    b2row = b2.reshape(1, NCLS)

    stab, sinit1 = _tc1(x_pad, W1, Ms, Md, Rexp)
    acc1 = _sc1(src_p, dst_p, stab)
    tab2, sinit2 = _tc2(acc1, sinit1, Rexp, W2tab, b1row)
    acc2 = _sc2(src_p, dst_p, tab2)
    out = _tc3(acc2, sinit2, b2row)
    return out[:N, :NCLS]


# speedup vs baseline: 116.9284x; 1.5761x over previous
import functools

import jax
import jax.numpy as jnp
from jax import lax
from jax.experimental import pallas as pl
from jax.experimental.pallas import tpu as pltpu
from jax.experimental.pallas import tpu_sc as plsc

N = 10000
D = 128
H1, C1 = 8, 8
F1 = H1 * C1
NCLS = 7
E = 320000

NT = 10112
BLK = 1264
NW = 32
K = 64
_CPW_RAW = -(-E // (K * NW))
CPW = _CPW_RAW + (_CPW_RAW % 2)
E_PAD = CPW * K * NW
RPS = NT // 16
ROW = 128


def _tc1_body(x_ref, w1_ref, ms_ref, md_ref, rexp_ref, stab_ref, sinit_ref):
    x = x_ref[...]
    h = jnp.dot(x, w1_ref[...], preferred_element_type=jnp.float32)
    a_s = jnp.dot(h, ms_ref[...], preferred_element_type=jnp.float32)
    a_d = jnp.dot(h, md_ref[...], preferred_element_type=jnp.float32)
    z8 = jnp.zeros((x.shape[0], 8), jnp.float32)
    z40 = jnp.zeros((x.shape[0], 40), jnp.float32)
    stab_ref[...] = jnp.concatenate([h, a_s, z8, a_d, z40], axis=1)
    e = a_s + a_d
    ee = jnp.exp(jnp.maximum(e, 0.2 * e))
    ee64 = jnp.dot(ee, rexp_ref[...], preferred_element_type=jnp.float32)
    sinit_ref[...] = jnp.concatenate([h * ee64, ee, z8], axis=1)


def _tc1(x_pad, W1, Ms, Md, Rexp):
    return pl.pallas_call(
        _tc1_body,
        grid=(NT // BLK,),
        in_specs=[
            pl.BlockSpec((BLK, D), lambda i: (i, 0)),
            pl.BlockSpec((D, F1), lambda i: (0, 0)),
            pl.BlockSpec((F1, 8), lambda i: (0, 0)),
            pl.BlockSpec((F1, 8), lambda i: (0, 0)),
            pl.BlockSpec((8, F1), lambda i: (0, 0)),
        ],
        out_specs=[
            pl.BlockSpec((BLK, ROW), lambda i: (i, 0)),
            pl.BlockSpec((BLK, 80), lambda i: (i, 0)),
        ],
        out_shape=[
            jax.ShapeDtypeStruct((NT, ROW), jnp.float32),
            jax.ShapeDtypeStruct((NT, 80), jnp.float32),
        ],
    )(x_pad, W1, Ms, Md, Rexp)


def _make_sc_body(edge_compute, unroll):
    def body(src_hbm, dst_hbm, stab_hbm, out_hbm,
             si0, si1, di0, di1, sx0, sx1, sbuf0, dbuf0, sbuf1, dbuf1,
             obuf0, obuf1, acc,
             is0, is1, gs0, gd0, gs1, gd1, ss0, ss1):
        cid = lax.axis_index("c")
        sid = lax.axis_index("s")
        wid = sid * 2 + cid
        z16 = jnp.zeros((16,), jnp.float32)

        @plsc.parallel_loop(0, K, unroll=2)
        def zrow(e):
            for c8 in range(8):
                obuf0[e, pl.ds(c8 * 16, 16)] = z16
                obuf1[e, pl.ds(c8 * 16, 16)] = z16
        for i in range(RPS // K):
            pltpu.sync_copy(obuf0, acc.at[pl.ds(sid * RPS + i * K, K)])
        rem = RPS % K
        if rem:
            pltpu.sync_copy(obuf0.at[pl.ds(0, rem)],
                            acc.at[pl.ds(sid * RPS + (RPS // K) * K, rem)])
        plsc.subcore_barrier()

        base0 = wid * CPW * K

        def start_idx(j, si, di, isem):
            pltpu.async_copy(src_hbm.at[pl.ds(base0 + j * K, K)], si, isem)
            pltpu.async_copy(dst_hbm.at[pl.ds(base0 + j * K, K)], di, isem)

        def wait_idx(j, si, di, isem):
            pltpu.make_async_copy(
                src_hbm.at[pl.ds(base0 + j * K, K)], si, isem).wait()
            pltpu.make_async_copy(
                dst_hbm.at[pl.ds(base0 + j * K, K)], di, isem).wait()

        def start_g(si, di, sb, db, gs, gd):
            pltpu.async_copy(stab_hbm.at[si], sb, gs)
            pltpu.async_copy(stab_hbm.at[di], db, gd)

        def wait_g(si, di, sb, db, gs, gd):
            pltpu.make_async_copy(stab_hbm.at[si], sb, gs).wait()
            pltpu.make_async_copy(stab_hbm.at[di], db, gd).wait()

        def compute(sb, db, ob):
            @plsc.parallel_loop(0, K, unroll=unroll)
            def edge(e):
                edge_compute(sb, db, ob, e)

        def wait_s(ob, sx, ss):
            pltpu.make_async_copy(ob, acc.at[sx], ss).wait()

        SI = (si0, si1)
        DI = (di0, di1)
        SX = (sx0, sx1)
        SB = (sbuf0, sbuf1)
        DB = (dbuf0, dbuf1)
        OB = (obuf0, obuf1)
        IS = (is0, is1)
        GS = (gs0, gs1)
        GD = (gd0, gd1)
        SS = (ss0, ss1)

        start_idx(0, si0, di0, is0)
        wait_idx(0, si0, di0, is0)
        start_g(si0, di0, sbuf0, dbuf0, gs0, gd0)
        start_idx(1, si1, di1, is1)

        def pipe(i, carry):
            j0 = i * 2
            for b in range(2):
                c = j0 + b
                o = 1 - b

                @pl.when(c + 1 < CPW)
                def _():
                    wait_idx(c + 1, SI[o], DI[o], IS[o])
                    start_g(SI[o], DI[o], SB[o], DB[o], GS[o], GD[o])

                wait_g(SI[b], DI[b], SB[b], DB[b], GS[b], GD[b])

                @pl.when(c >= 2)
                def _():
                    wait_s(OB[b], SX[b], SS[b])

                compute(SB[b], DB[b], OB[b])
                for q in range(K // 16):
                    SX[b][pl.ds(q * 16, 16)] = DI[b][pl.ds(q * 16, 16)]
                pltpu.async_copy(OB[b], acc.at[SX[b]], SS[b], add=True)

                @pl.when(c + 2 < CPW)
                def _():
                    start_idx(c + 2, SI[b], DI[b], IS[b])

            return carry

        lax.fori_loop(0, CPW // 2, pipe, 0)
        wait_s(obuf0, sx0, ss0)
        wait_s(obuf1, sx1, ss1)
        plsc.subcore_barrier()
        pltpu.sync_copy(acc.at[pl.ds(sid * RPS, RPS)],
                        out_hbm.at[cid, pl.ds(sid * RPS, RPS)])

    return body


def _sc_call(body):
    mesh = plsc.VectorSubcoreMesh(core_axis_name="c", subcore_axis_name="s")
    return functools.partial(
        pl.kernel,
        out_type=jax.ShapeDtypeStruct((2, NT, ROW), jnp.float32),
        mesh=mesh,
        scratch_types=[
            pltpu.VMEM((K,), jnp.int32),
            pltpu.VMEM((K,), jnp.int32),
            pltpu.VMEM((K,), jnp.int32),
            pltpu.VMEM((K,), jnp.int32),
            pltpu.VMEM((K,), jnp.int32),
            pltpu.VMEM((K,), jnp.int32),
            pltpu.VMEM((K, ROW), jnp.float32),
            pltpu.VMEM((K, ROW), jnp.float32),
            pltpu.VMEM((K, ROW), jnp.float32),
            pltpu.VMEM((K, ROW), jnp.float32),
            pltpu.VMEM((K, ROW), jnp.float32),
            pltpu.VMEM((K, ROW), jnp.float32),
            pltpu.MemorySpace.VMEM_SHARED((NT, ROW), jnp.float32),
            pltpu.SemaphoreType.DMA,
            pltpu.SemaphoreType.DMA,
            pltpu.SemaphoreType.DMA,
            pltpu.SemaphoreType.DMA,
            pltpu.SemaphoreType.DMA,
            pltpu.SemaphoreType.DMA,
            pltpu.SemaphoreType.DMA,
            pltpu.SemaphoreType.DMA,
        ],
    )(body)


def _edge1(lane, g01, g23, g45, g67):
    def f(sb, db, ob, e):
        s4 = sb[e, pl.ds(64, 16)]
        d4 = db[e, pl.ds(80, 16)]
        t = s4 + d4
        ee = jnp.exp(jnp.maximum(t, 0.2 * t))
        m01 = jnp.take_along_axis(ee, g01, axis=0, mode="promise_in_bounds")
        m23 = jnp.take_along_axis(ee, g23, axis=0, mode="promise_in_bounds")
        m45 = jnp.take_along_axis(ee, g45, axis=0, mode="promise_in_bounds")
        m67 = jnp.take_along_axis(ee, g67, axis=0, mode="promise_in_bounds")
        ob[e, pl.ds(0, 16)] = sb[e, pl.ds(0, 16)] * m01
        ob[e, pl.ds(16, 16)] = sb[e, pl.ds(16, 16)] * m23
        ob[e, pl.ds(32, 16)] = sb[e, pl.ds(32, 16)] * m45
        ob[e, pl.ds(48, 16)] = sb[e, pl.ds(48, 16)] * m67
        ob[e, pl.ds(64, 16)] = jnp.where(lane < 8, ee, 0.0)
    return f


def _sc1_body(src_hbm, dst_hbm, stab_hbm, out_hbm, *rest):
    lane = lax.iota(jnp.int32, 16)
    g01 = lane >> 3
    body = _make_sc_body(_edge1(lane, g01, g01 + 2, g01 + 4, g01 + 6),
                         unroll=4)
    return body(src_hbm, dst_hbm, stab_hbm, out_hbm, *rest)


def _sc1(src_p, dst_p, stab):
    return _sc_call(_sc1_body)(src_p, dst_p, stab)


def _tc2_body(acc_ref, sinit_ref, rexp_ref, w2t_ref, b1_ref,
              tab2_ref, sinit2_ref):
    a = acc_ref[0] + acc_ref[1]
    si = sinit_ref[...]
    num = a[:, 0:F1] + si[:, 0:F1]
    den = a[:, F1:F1 + 8] + si[:, F1:F1 + 8]
    dinv = 1.0 / (den + 1e-16)
    d64 = jnp.dot(dinv, rexp_ref[...], preferred_element_type=jnp.float32)
    out1 = num * d64 + b1_ref[...]
    x2 = jnp.where(out1 > 0, out1, jnp.exp(out1) - 1.0)
    t2 = jnp.dot(x2, w2t_ref[...], preferred_element_type=jnp.float32)
    tab2_ref[...] = t2
    a_s = t2[:, 7:8]
    a_d = t2[:, 8:9]
    e = a_s + a_d
    ee = jnp.exp(jnp.maximum(e, 0.2 * e))
    z = jnp.zeros((t2.shape[0], 8), jnp.float32)
    sinit2_ref[...] = jnp.concatenate([t2[:, 0:7] * ee, ee, z], axis=1)


def _tc2(acc1, sinit1, Rexp, W2tab, b1row):
    return pl.pallas_call(
        _tc2_body,
        grid=(NT // BLK,),
        in_specs=[
            pl.BlockSpec((2, BLK, ROW), lambda i: (0, i, 0)),
            pl.BlockSpec((BLK, 80), lambda i: (i, 0)),
            pl.BlockSpec((8, F1), lambda i: (0, 0)),
            pl.BlockSpec((F1, ROW), lambda i: (0, 0)),
            pl.BlockSpec((1, F1), lambda i: (0, 0)),
        ],
        out_specs=[
            pl.BlockSpec((BLK, ROW), lambda i: (i, 0)),
            pl.BlockSpec((BLK, 16), lambda i: (i, 0)),
        ],
        out_shape=[
            jax.ShapeDtypeStruct((NT, ROW), jnp.float32),
            jax.ShapeDtypeStruct((NT, 16), jnp.float32),
        ],
    )(acc1, sinit1, Rexp, W2tab, b1row)


def _edge2(lane, c7, c8v):
    def f(sb, db, ob, e):
        srow = sb[e, pl.ds(0, 16)]
        drow = db[e, pl.ds(0, 16)]
        a_s = jnp.take_along_axis(srow, c7, axis=0, mode="promise_in_bounds")
        a_d = jnp.take_along_axis(drow, c8v, axis=0, mode="promise_in_bounds")
        t = a_s + a_d
        ee = jnp.exp(jnp.maximum(t, 0.2 * t))
        o = jnp.where(lane < 7, srow * ee, jnp.where(lane == 7, ee, 0.0))
        ob[e, pl.ds(0, 16)] = o
    return f


def _sc2_body(src_hbm, dst_hbm, tab_hbm, out_hbm, *rest):
    lane = lax.iota(jnp.int32, 16)
    c7 = jnp.full((16,), 7, jnp.int32)
    c8v = jnp.full((16,), 8, jnp.int32)
    body = _make_sc_body(_edge2(lane, c7, c8v), unroll=8)
    return body(src_hbm, dst_hbm, tab_hbm, out_hbm, *rest)


def _sc2(src_p, dst_p, tab2):
    return _sc_call(_sc2_body)(src_p, dst_p, tab2)


def _tc3_body(acc_ref, sinit_ref, b2_ref, out_ref):
    a = acc_ref[0] + acc_ref[1]
    si = sinit_ref[...]
    num = a[:, 0:NCLS] + si[:, 0:NCLS]
    den = a[:, NCLS:NCLS + 1] + si[:, NCLS:NCLS + 1]
    logits = num / (den + 1e-16) + b2_ref[...]
    m = jnp.max(logits, axis=1, keepdims=True)
    s = logits - m
    lse = jnp.log(jnp.sum(jnp.exp(s), axis=1, keepdims=True))
    out_ref[...] = jnp.concatenate(
        [s - lse, jnp.zeros((a.shape[0], 1), jnp.float32)], axis=1)


def _tc3(acc2, sinit2, b2row):
    return pl.pallas_call(
        _tc3_body,
        grid=(NT // BLK,),
        in_specs=[
            pl.BlockSpec((2, BLK, ROW), lambda i: (0, i, 0)),
            pl.BlockSpec((BLK, 16), lambda i: (i, 0)),
            pl.BlockSpec((1, NCLS), lambda i: (0, 0)),
        ],
        out_specs=pl.BlockSpec((BLK, 8), lambda i: (i, 0)),
        out_shape=jax.ShapeDtypeStruct((NT, 8), jnp.float32),
    )(acc2, sinit2, b2row)


def kernel(x, edge_index, W1, a_src1, a_dst1, b1, W2, a_src2, a_dst2, b2):
    f32 = jnp.float32
    x_pad = jnp.concatenate([x, jnp.zeros((NT - N, D), f32)], axis=0)

    eye8 = jnp.eye(8, dtype=f32)
    Rexp = jnp.kron(eye8, jnp.ones((1, 8), f32))
    Ms = Rexp.T * a_src1.reshape(F1, 1)
    Md = Rexp.T * a_dst1.reshape(F1, 1)

    w2s = W2 @ a_src2.reshape(NCLS, 1)
    w2d = W2 @ a_dst2.reshape(NCLS, 1)
    W2tab = jnp.concatenate(
        [W2, w2s, w2d, jnp.zeros((F1, ROW - 9), f32)], axis=1)

    src = edge_index[0].astype(jnp.int32)
    dst = edge_index[1].astype(jnp.int32)
    pad_idx = N + (jnp.arange(E_PAD - E, dtype=jnp.int32) % 16)
    src_p = jnp.concatenate([src, pad_idx])
    dst_p = jnp.concatenate([dst, pad_idx])
    src_p, dst_p = jax.lax.optimization_barrier((src_p, dst_p))

    b1row = b1.reshape(1, F1)
    b2row = b2.reshape(1, NCLS)

    stab, sinit1 = _tc1(x_pad, W1, Ms, Md, Rexp)
    acc1 = _sc1(src_p, dst_p, stab)
    tab2, sinit2 = _tc2(acc1, sinit1, Rexp, W2tab, b1row)
    acc2 = _sc2(src_p, dst_p, tab2)
    out = _tc3(acc2, sinit2, b2row)
    return out[:N, :NCLS]
